# Initial kernel scaffold; baseline (speedup 1.0000x reference)
import jax
import jax.numpy as jnp
from jax.experimental import pallas as pl


def kernel(positions, node_attrs, edge_index, shifts, batch, head, params):
    raise NotImplementedError("write your pallas kernel here")


# speedup vs baseline: 1.2530x; 1.2530x over previous
import functools

import jax
import jax.numpy as jnp
from jax import lax
from jax.experimental import pallas as pl
from jax.experimental.pallas import tpu as pltpu
from jax.experimental.pallas import tpu_sc as plsc

N = 10000
E = 320000
C = 128
NEA = 10
G = 16
NB = 8
RMAX = 5.0
PPOW = 6
AVG = 32.0

NW = 32
PERW = E // NW
CH = 80
NCH = PERW // CH
NRC = N // CH
BN = 1000
BE = 1600
COEF = (2.0 / RMAX) ** 0.5
PI = 3.14159265358979323846

_mesh = plsc.VectorSubcoreMesh(core_axis_name="c", subcore_axis_name="s")


def _wid():
    return lax.axis_index("s") * 2 + lax.axis_index("c")


def _sc_gather(table, idx3, d):
    @functools.partial(
        pl.kernel, mesh=_mesh,
        out_type=jax.ShapeDtypeStruct((E, d), jnp.float32),
        scratch_types=[pltpu.VMEM((NCH, CH), jnp.int32),
                       pltpu.VMEM((CH, d), jnp.float32),
                       pltpu.SemaphoreType.DMA])
    def k(table_hbm, idx_hbm, out_hbm, idx_v, rows_v, sem):
        w = _wid()
        pltpu.sync_copy(idx_hbm.at[w], idx_v)

        def body(j, carry):
            pltpu.async_copy(table_hbm.at[idx_v.at[j]], rows_v, sem).wait()
            pltpu.sync_copy(rows_v, out_hbm.at[pl.ds(w * PERW + j * CH, CH)])
            return carry

        lax.fori_loop(0, NCH, body, 0)

    return k(table, idx3)


def _sc_posdiff(tab, sidx3, ridx3):
    @functools.partial(
        pl.kernel, mesh=_mesh,
        out_type=jax.ShapeDtypeStruct((E, 16), jnp.float32),
        scratch_types=[pltpu.VMEM((NCH, CH), jnp.int32),
                       pltpu.VMEM((NCH, CH), jnp.int32),
                       pltpu.VMEM((CH, 128), jnp.float32),
                       pltpu.VMEM((CH, 128), jnp.float32),
                       pltpu.VMEM((CH, 16), jnp.float32),
                       pltpu.SemaphoreType.DMA])
    def k(tab_hbm, si_hbm, ri_hbm, out_hbm, si_v, ri_v, bufa, bufb, bufc,
          sem):
        w = _wid()
        pltpu.sync_copy(si_hbm.at[w], si_v)
        pltpu.sync_copy(ri_hbm.at[w], ri_v)

        def body(j, carry):
            pltpu.async_copy(tab_hbm.at[si_v.at[j]], bufa, sem).wait()
            pltpu.async_copy(tab_hbm.at[ri_v.at[j]], bufb, sem).wait()

            def sub(r, c2):
                bufc[r, :] = bufb[r, pl.ds(0, 16)] - bufa[r, pl.ds(0, 16)]
                return c2

            lax.fori_loop(0, CH, sub, 0)
            pltpu.sync_copy(bufc, out_hbm.at[pl.ds(w * PERW + j * CH, CH)])
            return carry

        lax.fori_loop(0, NCH, body, 0)

    return k(tab, sidx3, ridx3)


def _sc_scatter(vals, idx3, d):
    @functools.partial(
        pl.kernel, mesh=_mesh,
        out_type=jax.ShapeDtypeStruct((2, N, d), jnp.float32),
        scratch_types=[pltpu.VMEM((CH,), jnp.int32),
                       pltpu.VMEM((CH, d), jnp.float32),
                       pltpu.VMEM_SHARED((N, d), jnp.float32),
                       pltpu.SemaphoreType.DMA])
    def k(vals_hbm, idx_hbm, out_hbm, idx_v, rows_v, acc, sem):
        cid = lax.axis_index("c")
        sid = lax.axis_index("s")
        w = sid * 2 + cid

        def zrow(i, carry):
            r = i // (d // 16)
            q = (i % (d // 16)) * 16
            rows_v[r, pl.ds(q, 16)] = jnp.zeros((16,), jnp.float32)
            return carry

        lax.fori_loop(0, CH * (d // 16), zrow, 0)

        def zacc(kk, carry):
            @pl.when(kk % 16 == sid)
            def _():
                pltpu.sync_copy(rows_v, acc.at[pl.ds(kk * CH, CH)])
            return carry

        lax.fori_loop(0, NRC, zacc, 0)
        plsc.subcore_barrier()

        def body(j, carry):
            pltpu.sync_copy(idx_hbm.at[w, j], idx_v)
            pltpu.sync_copy(vals_hbm.at[pl.ds(w * PERW + j * CH, CH)], rows_v)
            pltpu.sync_copy(rows_v, acc.at[idx_v], add=True)
            return carry

        lax.fori_loop(0, NCH, body, 0)
        plsc.subcore_barrier()

        def cout(kk, carry):
            @pl.when(kk % 16 == sid)
            def _():
                pltpu.sync_copy(acc.at[pl.ds(kk * CH, CH)],
                                out_hbm.at[cid, pl.ds(kk * CH, CH)])
            return carry

        lax.fori_loop(0, NRC, cout, 0)

    return k(vals, idx3)


def _sc_force(gv, sidx3, ridx3):
    @functools.partial(
        pl.kernel, mesh=_mesh,
        out_type=jax.ShapeDtypeStruct((2, N, 128), jnp.float32),
        scratch_types=[pltpu.VMEM((CH,), jnp.int32),
                       pltpu.VMEM((CH, 16), jnp.float32),
                       pltpu.VMEM((CH, 128), jnp.float32),
                       pltpu.VMEM_SHARED((N, 128), jnp.float32),
                       pltpu.SemaphoreType.DMA])
    def k(gv_hbm, si_hbm, ri_hbm, out_hbm, idx_v, rows16, rows_v, acc, sem):
        cid = lax.axis_index("c")
        sid = lax.axis_index("s")
        w = sid * 2 + cid

        def zrow(i, carry):
            r = i // 8
            q = (i % 8) * 16
            rows_v[r, pl.ds(q, 16)] = jnp.zeros((16,), jnp.float32)
            return carry

        lax.fori_loop(0, CH * 8, zrow, 0)

        def zacc(kk, carry):
            @pl.when(kk % 16 == sid)
            def _():
                pltpu.sync_copy(rows_v, acc.at[pl.ds(kk * CH, CH)])
            return carry

        lax.fori_loop(0, NRC, zacc, 0)
        plsc.subcore_barrier()

        def body(j, carry):
            pltpu.sync_copy(gv_hbm.at[pl.ds(w * PERW + j * CH, CH)], rows16)
            pltpu.sync_copy(si_hbm.at[w, j], idx_v)

            def cp(r, c2):
                rows_v[r, pl.ds(0, 16)] = rows16[r, :]
                return c2

            lax.fori_loop(0, CH, cp, 0)
            pltpu.sync_copy(rows_v, acc.at[idx_v], add=True)

            def neg(r, c2):
                rows_v[r, pl.ds(0, 16)] = -rows16[r, :]
                return c2

            lax.fori_loop(0, CH, neg, 0)
            pltpu.sync_copy(ri_hbm.at[w, j], idx_v)
            pltpu.sync_copy(rows_v, acc.at[idx_v], add=True)
            return carry

        lax.fori_loop(0, NCH, body, 0)
        plsc.subcore_barrier()

        def cout(kk, carry):
            @pl.when(kk % 16 == sid)
            def _():
                pltpu.sync_copy(acc.at[pl.ds(kk * CH, CH)],
                                out_hbm.at[cid, pl.ds(kk * CH, CH)])
            return carry

        lax.fori_loop(0, NRC, cout, 0)

    return k(gv, sidx3, ridx3)


def _sigmoid(x):
    return 1.0 / (1.0 + jnp.exp(-x))


def _silu(x):
    return x * _sigmoid(x)


def _dsilu(x):
    s = _sigmoid(x)
    return s + x * s * (1.0 - s)


def _dot(a, b):
    return jnp.dot(a, b, preferred_element_type=jnp.float32)


def _radial_pair(l):
    r = jnp.maximum(l, 1e-6)
    nrow = (lax.broadcasted_iota(jnp.int32, (l.shape[0], NB), 1)
            .astype(jnp.float32) + 1.0)
    arg = nrow * (PI / RMAX) * r
    s = jnp.sin(arg)
    cc = jnp.cos(arg)
    bes = COEF * s / r
    uu = r * (1.0 / RMAX)
    u2 = uu * uu
    u3 = u2 * uu
    u5 = u2 * u3
    u6 = u3 * u3
    u7 = u6 * uu
    u8 = u7 * uu
    mask = (uu < 1.0).astype(jnp.float32)
    env = (1.0 - 28.0 * u6 + 48.0 * u7 - 21.0 * u8) * mask
    fmat = bes * env
    dbes = COEF * (nrow * (PI / RMAX) * cc / r - s / (r * r))
    om = 1.0 - uu
    denv = -168.0 * u5 * om * om * mask * (1.0 / RMAX)
    dclip = jnp.where(l > 1e-6, 1.0, 0.5)
    dfmat = (dbes * env + bes * denv) * dclip
    return fmat, dfmat


def _sh_feats(x, y, z):
    one = jnp.ones_like(x)
    return jnp.concatenate(
        [one, x, y, z, x * x, y * y, z * z, x * y, x * z, y * z,
         x * x * x, y * y * y, z * z * z, x * x * y, x * x * z,
         y * y * x, y * y * z, z * z * x, z * z * y, x * y * z], axis=1)


def _edge_grid(n_in_cols, n_out_cols):
    grid = (E // BE,)
    ins = [pl.BlockSpec((BE, c), lambda i: (i, 0)) for c in n_in_cols]
    outs = [pl.BlockSpec((BE, c), lambda i: (i, 0)) for c in n_out_cols]
    return grid, ins, outs


def _w_spec(shape):
    return pl.BlockSpec(shape, lambda i: tuple(0 for _ in shape))


def _tc_node_pre(na, we, aet, w1a, wata, wska, watb):
    def body(na_ref, we_ref, aet_ref, w1a_ref, wata_ref, wska_ref, watb_ref,
             h0_ref, y1_ref, sk1_ref, a2_ref, ne0_ref):
        na_b = na_ref[...]
        h0 = _dot(na_b, we_ref[...])
        h0_ref[...] = h0
        y1_ref[...] = _dot(h0, w1a_ref[...])
        a1 = _dot(na_b, wata_ref[...])
        sk1_ref[...] = _dot(h0 * a1, wska_ref[...])
        a2_ref[...] = _dot(na_b, watb_ref[...])
        ne0_ref[...] = _dot(na_b, aet_ref[...])

    f = pl.pallas_call(
        body,
        grid=(N // BN,),
        in_specs=[pl.BlockSpec((BN, NEA), lambda i: (i, 0)),
                  _w_spec((NEA, C)), _w_spec((NEA, 1)), _w_spec((C, C)),
                  _w_spec((NEA, C)), _w_spec((C, C)), _w_spec((NEA, C))],
        out_specs=[pl.BlockSpec((BN, C), lambda i: (i, 0)),
                   pl.BlockSpec((BN, C), lambda i: (i, 0)),
                   pl.BlockSpec((BN, C), lambda i: (i, 0)),
                   pl.BlockSpec((BN, C), lambda i: (i, 0)),
                   pl.BlockSpec((BN, 1), lambda i: (i, 0))],
        out_shape=[jax.ShapeDtypeStruct((N, C), jnp.float32)] * 4
        + [jax.ShapeDtypeStruct((N, 1), jnp.float32)],
    )
    return f(na, we, aet, w1a, wata, wska, watb)


def _tc_geom(dmat, shifts, wr1a, wr2a, wr1b, wr2b, wsha, wshb):
    def body(d_ref, sh_ref, wr1a_ref, wr2a_ref, wr1b_ref, wr2b_ref,
             wsha_ref, wshb_ref, ge_ref, f_ref, rwa_ref, rwb_ref):
        d = d_ref[...]
        v = d[:, 0:3] + sh_ref[...]
        l = jnp.sqrt(jnp.sum(v * v, axis=1, keepdims=True) + 1e-12)
        u = v / l
        x, y, z = u[:, 0:1], u[:, 1:2], u[:, 2:3]
        sh20 = _sh_feats(x, y, z)
        ssa = _dot(sh20, wsha_ref[...])
        ssb = _dot(sh20, wshb_ref[...])
        fmat, _ = _radial_pair(l)
        f_ref[...] = fmat
        zero = jnp.zeros_like(l)
        ge_ref[...] = jnp.concatenate([v, l, ssa, ssb, zero, zero], axis=1)
        rwa_ref[...] = _dot(_silu(_dot(fmat, wr1a_ref[...])), wr2a_ref[...])
        rwb_ref[...] = _dot(_silu(_dot(fmat, wr1b_ref[...])), wr2b_ref[...])

    grid, _, _ = _edge_grid([], [])
    f = pl.pallas_call(
        body,
        grid=grid,
        in_specs=[pl.BlockSpec((BE, 16), lambda i: (i, 0)),
                  pl.BlockSpec((BE, 3), lambda i: (i, 0)),
                  _w_spec((NB, 64)), _w_spec((64, C)),
                  _w_spec((NB, 64)), _w_spec((64, C)),
                  _w_spec((20, 1)), _w_spec((20, 1))],
        out_specs=[pl.BlockSpec((BE, 8), lambda i: (i, 0)),
                   pl.BlockSpec((BE, NB), lambda i: (i, 0)),
                   pl.BlockSpec((BE, C), lambda i: (i, 0)),
                   pl.BlockSpec((BE, C), lambda i: (i, 0))],
        out_shape=[jax.ShapeDtypeStruct((E, 8), jnp.float32),
                   jax.ShapeDtypeStruct((E, NB), jnp.float32),
                   jax.ShapeDtypeStruct((E, C), jnp.float32),
                   jax.ShapeDtypeStruct((E, C), jnp.float32)],
    )
    return f(dmat, shifts, wr1a, wr2a, wr1b, wr2b, wsha, wshb)


def _tc_msg(ys, rw, ge, col):
    def body(ys_ref, rw_ref, ge_ref, out_ref):
        ss = ge_ref[...][:, col:col + 1]
        out_ref[...] = ys_ref[...] * rw_ref[...] * ss

    f = pl.pallas_call(
        body,
        grid=(E // BE,),
        in_specs=[pl.BlockSpec((BE, C), lambda i: (i, 0)),
                  pl.BlockSpec((BE, C), lambda i: (i, 0)),
                  pl.BlockSpec((BE, 8), lambda i: (i, 0))],
        out_specs=[pl.BlockSpec((BE, C), lambda i: (i, 0))],
        out_shape=[jax.ShapeDtypeStruct((E, C), jnp.float32)],
    )
    return f(ys, rw, ge)[0]


def _tc_node_mid(s1a, s1b, sk1, a2, wouta, wro, w1b, wskb):
    def body(sa_ref, sb_ref, sk1_ref, a2_ref, wouta_ref, wro_ref, w1b_ref,
             wskb_ref, y2_ref, sk2_ref, ne1_ref):
        agg = (sa_ref[...] + sb_ref[...]) * (1.0 / AVG)
        h1 = _dot(agg, wouta_ref[...]) + sk1_ref[...]
        ne1_ref[...] = _dot(h1, wro_ref[...])
        y2_ref[...] = _dot(h1, w1b_ref[...])
        sk2_ref[...] = _dot(h1 * a2_ref[...], wskb_ref[...])

    f = pl.pallas_call(
        body,
        grid=(N // BN,),
        in_specs=[pl.BlockSpec((BN, C), lambda i: (i, 0))] * 4
        + [_w_spec((C, C)), _w_spec((C, 1)), _w_spec((C, C)), _w_spec((C, C))],
        out_specs=[pl.BlockSpec((BN, C), lambda i: (i, 0)),
                   pl.BlockSpec((BN, C), lambda i: (i, 0)),
                   pl.BlockSpec((BN, 1), lambda i: (i, 0))],
        out_shape=[jax.ShapeDtypeStruct((N, C), jnp.float32),
                   jax.ShapeDtypeStruct((N, C), jnp.float32),
                   jax.ShapeDtypeStruct((N, 1), jnp.float32)],
    )
    return f(s1a, s1b, sk1, a2, wouta, wro, w1b, wskb)


def _tc_node_post(s2a, s2b, sk2, a2, woutb, wm1, wm2c, wm2r, wm1t, woutbt,
                  wskbt, wror):
    def body(sa_ref, sb_ref, sk2_ref, a2_ref, woutb_ref, wm1_ref, wm2c_ref,
             wm2r_ref, wm1t_ref, woutbt_ref, wskbt_ref, wror_ref,
             ne2_ref, ga2_ref, gh1p_ref):
        agg = (sa_ref[...] + sb_ref[...]) * (1.0 / AVG)
        h2 = _dot(agg, woutb_ref[...]) + sk2_ref[...]
        z2 = _dot(h2, wm1_ref[...])
        ne2_ref[...] = _dot(_silu(z2), wm2c_ref[...])
        gz = _dsilu(z2) * wm2r_ref[...]
        gh2 = _dot(gz, wm1t_ref[...])
        ga2_ref[...] = _dot(gh2, woutbt_ref[...]) * (1.0 / AVG)
        gh1p_ref[...] = (_dot(gh2, wskbt_ref[...]) * a2_ref[...]
                         + wror_ref[...])

    f = pl.pallas_call(
        body,
        grid=(N // BN,),
        in_specs=[pl.BlockSpec((BN, C), lambda i: (i, 0))] * 4
        + [_w_spec((C, C)), _w_spec((C, 16)), _w_spec((16, 1)),
           _w_spec((1, 16)), _w_spec((16, C)), _w_spec((C, C)),
           _w_spec((C, C)), _w_spec((1, C))],
        out_specs=[pl.BlockSpec((BN, 1), lambda i: (i, 0)),
                   pl.BlockSpec((BN, C), lambda i: (i, 0)),
                   pl.BlockSpec((BN, C), lambda i: (i, 0))],
        out_shape=[jax.ShapeDtypeStruct((N, 1), jnp.float32),
                   jax.ShapeDtypeStruct((N, C), jnp.float32),
                   jax.ShapeDtypeStruct((N, C), jnp.float32)],
    )
    return f(s2a, s2b, sk2, a2, woutb, wm1, wm2c, wm2r, wm1t, woutbt, wskbt,
             wror)


def _tc_edge_bwd2(gmsg, ys, rw, ge, fmat, wr1b, wr2bt, wr1bt):
    def body(gmsg_ref, ys_ref, rw_ref, ge_ref, f_ref, wr1b_ref, wr2bt_ref,
             wr1bt_ref, gys_ref, eb_ref):
        gmsg_b = gmsg_ref[...]
        ys_b = ys_ref[...]
        rw_b = rw_ref[...]
        ge = ge_ref[...]
        ssb = ge[:, 5:6]
        l = ge[:, 3:4]
        t = gmsg_b * ys_b
        gys_ref[...] = gmsg_b * rw_b * ssb
        gss = jnp.sum(t * rw_b, axis=1, keepdims=True)
        grw = t * ssb
        fmat_b = f_ref[...]
        z = _dot(fmat_b, wr1b_ref[...])
        gq = _dot(grw, wr2bt_ref[...]) * _dsilu(z)
        gf = _dot(gq, wr1bt_ref[...])
        _, dfdl = _radial_pair(l)
        gl = jnp.sum(gf * dfdl, axis=1, keepdims=True)
        zero = jnp.zeros_like(gl)
        eb_ref[...] = jnp.concatenate(
            [gl, gss, zero, zero, zero, zero, zero, zero], axis=1)

    f = pl.pallas_call(
        body,
        grid=(E // BE,),
        in_specs=[pl.BlockSpec((BE, C), lambda i: (i, 0)),
                  pl.BlockSpec((BE, C), lambda i: (i, 0)),
                  pl.BlockSpec((BE, C), lambda i: (i, 0)),
                  pl.BlockSpec((BE, 8), lambda i: (i, 0)),
                  pl.BlockSpec((BE, NB), lambda i: (i, 0)),
                  _w_spec((NB, 64)), _w_spec((C, 64)), _w_spec((64, NB))],
        out_specs=[pl.BlockSpec((BE, C), lambda i: (i, 0)),
                   pl.BlockSpec((BE, 8), lambda i: (i, 0))],
        out_shape=[jax.ShapeDtypeStruct((E, C), jnp.float32),
                   jax.ShapeDtypeStruct((E, 8), jnp.float32)],
    )
    return f(gmsg, ys, rw, ge, fmat, wr1b, wr2bt, wr1bt)


def _tc_node_gh1(gh1p, ya, yb, w1bt, woutat):
    def body(gh1p_ref, ya_ref, yb_ref, w1bt_ref, woutat_ref, ga1_ref):
        gh1 = gh1p_ref[...] + _dot(ya_ref[...] + yb_ref[...], w1bt_ref[...])
        ga1_ref[...] = _dot(gh1, woutat_ref[...]) * (1.0 / AVG)

    f = pl.pallas_call(
        body,
        grid=(N // BN,),
        in_specs=[pl.BlockSpec((BN, C), lambda i: (i, 0))] * 3
        + [_w_spec((C, C)), _w_spec((C, C))],
        out_specs=[pl.BlockSpec((BN, C), lambda i: (i, 0))],
        out_shape=[jax.ShapeDtypeStruct((N, C), jnp.float32)],
    )
    return f(gh1p, ya, yb, w1bt, woutat)[0]


def _tc_edge_bwd1(gmsg, ys, rw, ge, fmat, eb2, wr1a, wr2at, wr1at, wshar,
                  wshbr):
    def body(gmsg_ref, ys_ref, rw_ref, ge_ref, f_ref, eb2_ref, wr1a_ref,
             wr2at_ref, wr1at_ref, wshar_ref, wshbr_ref, gv_ref):
        gmsg_b = gmsg_ref[...]
        ys_b = ys_ref[...]
        rw_b = rw_ref[...]
        ge = ge_ref[...]
        eb2 = eb2_ref[...]
        v = ge[:, 0:3]
        l = ge[:, 3:4]
        ssa = ge[:, 4:5]
        t = gmsg_b * ys_b
        gsa = jnp.sum(t * rw_b, axis=1, keepdims=True)
        grw = t * ssa
        fmat_b = f_ref[...]
        z = _dot(fmat_b, wr1a_ref[...])
        gq = _dot(grw, wr2at_ref[...]) * _dsilu(z)
        gf = _dot(gq, wr1at_ref[...])
        _, dfdl = _radial_pair(l)
        gl = jnp.sum(gf * dfdl, axis=1, keepdims=True) + eb2[:, 0:1]
        gsb = eb2[:, 1:2]
        c20 = gsa * wshar_ref[...] + gsb * wshbr_ref[...]
        u = v / l
        x, y, z3 = u[:, 0:1], u[:, 1:2], u[:, 2:3]
        gx = (c20[:, 1:2] + 2.0 * x * c20[:, 4:5] + y * c20[:, 7:8]
              + z3 * c20[:, 8:9] + 3.0 * x * x * c20[:, 10:11]
              + 2.0 * x * y * c20[:, 13:14] + 2.0 * x * z3 * c20[:, 14:15]
              + y * y * c20[:, 15:16] + z3 * z3 * c20[:, 17:18]
              + y * z3 * c20[:, 19:20])
        gy = (c20[:, 2:3] + 2.0 * y * c20[:, 5:6] + x * c20[:, 7:8]
              + z3 * c20[:, 9:10] + 3.0 * y * y * c20[:, 11:12]
              + x * x * c20[:, 13:14] + 2.0 * x * y * c20[:, 15:16]
              + 2.0 * y * z3 * c20[:, 16:17] + z3 * z3 * c20[:, 18:19]
              + x * z3 * c20[:, 19:20])
        gz = (c20[:, 3:4] + 2.0 * z3 * c20[:, 6:7] + x * c20[:, 8:9]
              + y * c20[:, 9:10] + 3.0 * z3 * z3 * c20[:, 12:13]
              + x * x * c20[:, 14:15] + y * y * c20[:, 16:17]
              + 2.0 * z3 * x * c20[:, 17:18] + 2.0 * z3 * y * c20[:, 18:19]
              + x * y * c20[:, 19:20])
        gu = jnp.concatenate([gx, gy, gz], axis=1)
        gudotu = jnp.sum(gu * u, axis=1, keepdims=True)
        gv = gl * u + (gu - gudotu * u) / l
        pad = jnp.zeros((gv.shape[0], 13), jnp.float32)
        gv_ref[...] = jnp.concatenate([gv, pad], axis=1)

    f = pl.pallas_call(
        body,
        grid=(E // BE,),
        in_specs=[pl.BlockSpec((BE, C), lambda i: (i, 0)),
                  pl.BlockSpec((BE, C), lambda i: (i, 0)),
                  pl.BlockSpec((BE, C), lambda i: (i, 0)),
                  pl.BlockSpec((BE, 8), lambda i: (i, 0)),
                  pl.BlockSpec((BE, NB), lambda i: (i, 0)),
                  pl.BlockSpec((BE, 8), lambda i: (i, 0)),
                  _w_spec((NB, 64)), _w_spec((C, 64)), _w_spec((64, NB)),
                  _w_spec((1, 20)), _w_spec((1, 20))],
        out_specs=[pl.BlockSpec((BE, 16), lambda i: (i, 0))],
        out_shape=[jax.ShapeDtypeStruct((E, 16), jnp.float32)],
    )
    return f(gmsg, ys, rw, ge, fmat, eb2, wr1a, wr2at, wr1at, wshar, wshbr)[0]


def _tc_finalize(gpa, gpb, ne0, ne1, ne2, batch_row):
    def body(gpa_ref, gpb_ref, ne0_ref, ne1_ref, ne2_ref, b_ref,
             forces_ref, contrib_ref, tot_ref):
        i = pl.program_id(0)
        forces_ref[...] = gpa_ref[...][:, 0:3] + gpb_ref[...][:, 0:3]
        brow = b_ref[...].reshape(1, BN)
        onehot_t = (lax.broadcasted_iota(jnp.int32, (G, BN), 0)
                    == brow).astype(jnp.float32)
        necat = jnp.concatenate([ne0_ref[...], ne1_ref[...], ne2_ref[...]],
                                axis=1)
        cblk = jnp.dot(onehot_t, necat, preferred_element_type=jnp.float32,
                       precision=lax.Precision.HIGHEST)
        tblk = jnp.sum(cblk, axis=1, keepdims=True)

        @pl.when(i == 0)
        def _():
            contrib_ref[...] = cblk
            tot_ref[...] = tblk

        @pl.when(i > 0)
        def _():
            contrib_ref[...] = contrib_ref[...] + cblk
            tot_ref[...] = tot_ref[...] + tblk

    f = pl.pallas_call(
        body,
        grid=(N // BN,),
        in_specs=[pl.BlockSpec((BN, 128), lambda i: (i, 0)),
                  pl.BlockSpec((BN, 128), lambda i: (i, 0)),
                  pl.BlockSpec((BN, 1), lambda i: (i, 0)),
                  pl.BlockSpec((BN, 1), lambda i: (i, 0)),
                  pl.BlockSpec((BN, 1), lambda i: (i, 0)),
                  pl.BlockSpec((1, 1, BN), lambda i: (i, 0, 0))],
        out_specs=[pl.BlockSpec((BN, 3), lambda i: (i, 0)),
                   pl.BlockSpec((G, 3), lambda i: (0, 0)),
                   pl.BlockSpec((G, 1), lambda i: (0, 0))],
        out_shape=[jax.ShapeDtypeStruct((N, 3), jnp.float32),
                   jax.ShapeDtypeStruct((G, 3), jnp.float32),
                   jax.ShapeDtypeStruct((G, 1), jnp.float32)],
    )
    return f(gpa, gpb, ne0, ne1, ne2, batch_row)


def kernel(positions, node_attrs, edge_index, shifts, batch, head, params):
    del head
    pos = positions.astype(jnp.float32)
    na = node_attrs.astype(jnp.float32)
    send = edge_index[0].astype(jnp.int32)
    recv = edge_index[1].astype(jnp.int32)
    send3 = send.reshape(NW, NCH, CH)
    recv3 = recv.reshape(NW, NCH, CH)
    pos_p = jnp.zeros((N, 128), jnp.float32).at[:, 0:3].set(pos)
    batch_row = batch.astype(jnp.int32).reshape(N // BN, 1, BN)

    la, lb = params['layers']
    aet = params['atomic_energies'].astype(jnp.float32).T
    wro = params['w_ro0'].reshape(C, 1)
    wror = params['w_ro0'].reshape(1, C)
    wm2c = params['wm2'].reshape(16, 1)
    wm2r = params['wm2'].reshape(1, 16)

    dmat = _sc_posdiff(pos_p, send3, recv3)
    ge, fmat, rwa, rwb = _tc_geom(
        dmat, shifts.astype(jnp.float32), la['Wr1'], la['Wr2'], lb['Wr1'],
        lb['Wr2'], la['w_sh'].reshape(20, 1), lb['w_sh'].reshape(20, 1))
    h0, y1, sk1, a2, ne0 = _tc_node_pre(
        na, params['W_embed'], aet, la['W1'], la['W_attr'], la['W_skip'],
        lb['W_attr'])
    del h0
    y1s = _sc_gather(y1, send3, C)
    msga = _tc_msg(y1s, rwa, ge, 4)
    s1 = _sc_scatter(msga, recv3, C)
    y2, sk2, ne1 = _tc_node_mid(s1[0], s1[1], sk1, a2, la['W_out'], wro,
                                lb['W1'], lb['W_skip'])
    y2s = _sc_gather(y2, send3, C)
    msgb = _tc_msg(y2s, rwb, ge, 5)
    s2 = _sc_scatter(msgb, recv3, C)
    ne2, ga2, gh1p = _tc_node_post(
        s2[0], s2[1], sk2, a2, lb['W_out'], params['Wm1'], wm2c, wm2r,
        params['Wm1'].T, lb['W_out'].T, lb['W_skip'].T, wror)
    gmsg2 = _sc_gather(ga2, recv3, C)
    gys2, eb2 = _tc_edge_bwd2(gmsg2, y2s, rwb, ge, fmat, lb['Wr1'],
                              lb['Wr2'].T, lb['Wr1'].T)
    gy2 = _sc_scatter(gys2, send3, C)
    ga1 = _tc_node_gh1(gh1p, gy2[0], gy2[1], lb['W1'].T, la['W_out'].T)
    gmsg1 = _sc_gather(ga1, recv3, C)
    gv = _tc_edge_bwd1(gmsg1, y1s, rwa, ge, fmat, eb2, la['Wr1'],
                       la['Wr2'].T, la['Wr1'].T, la['w_sh'].reshape(1, 20),
                       lb['w_sh'].reshape(1, 20))
    gp = _sc_force(gv, send3, recv3)
    forces, contrib, tot = _tc_finalize(gp[0], gp[1], ne0, ne1, ne2,
                                        batch_row)
    return tot[:, 0], contrib, forces
